# Initial kernel scaffold; baseline (speedup 1.0000x reference)
import jax
import jax.numpy as jnp
from jax.experimental import pallas as pl


def kernel(source_pc, target_pc):
    raise NotImplementedError("write your pallas kernel here")


# speedup vs baseline: 14.0730x; 14.0730x over previous
import jax
import jax.numpy as jnp
from jax import lax
from jax.experimental import pallas as pl
from jax.experimental.pallas import tpu as pltpu

_K = 16
_BM = 256


def _knn_body(q_ref, rt_ref, num_ref, cnt_ref, d_scr):
    rx = rt_ref[0:1, :]
    ry = rt_ref[1:2, :]
    rz = rt_ref[2:3, :]
    qx = q_ref[:, 0:1]
    qy = q_ref[:, 1:2]
    qz = q_ref[:, 2:3]

    r2 = rx * rx + ry * ry + rz * rz
    rvalid = (rx != 0.0) | (ry != 0.0) | (rz != 0.0)
    q2 = qx * qx + qy * qy + qz * qz
    qvalid = (qx != 0.0) | (qy != 0.0) | (qz != 0.0)

    bf = jnp.bfloat16
    f32 = jnp.float32
    qxb, qyb, qzb = (qx.astype(bf).astype(f32), qy.astype(bf).astype(f32),
                     qz.astype(bf).astype(f32))
    rxb, ryb, rzb = (rx.astype(bf).astype(f32), ry.astype(bf).astype(f32),
                     rz.astype(bf).astype(f32))
    d2 = (q2 + r2) - 2.0 * (qxb * rxb + qyb * ryb + qzb * rzb)
    d_scr[:, :] = jnp.where(rvalid, d2, jnp.inf)

    def step(_, carry):
        acc, rem = carry
        d = d_scr[:, :]
        v = jnp.min(d, axis=1, keepdims=True)
        eq = d == v
        c = jnp.sum(eq.astype(jnp.float32), axis=1, keepdims=True)
        take = jnp.minimum(c, rem)
        dist = jnp.sqrt(jnp.maximum(v, 0.0))
        acc = acc + jnp.where(take > 0.0, take * dist, 0.0)
        d_scr[:, :] = jnp.where(eq, jnp.inf, d)
        return acc, rem - take

    bm = q_ref.shape[0]
    acc0 = jnp.zeros((bm, 1), jnp.float32)
    rem0 = jnp.full((bm, 1), float(_K), jnp.float32)
    acc, _ = lax.fori_loop(0, _K, step, (acc0, rem0))

    num_ref[...] = jnp.broadcast_to(jnp.sum(jnp.where(qvalid, acc, 0.0)), (1, 1, 1))
    cnt_ref[...] = jnp.broadcast_to(jnp.sum(qvalid.astype(jnp.float32)), (1, 1, 1))


def kernel(source_pc, target_pc):
    q = source_pc.reshape(-1, source_pc.shape[-1])
    r = target_pc.reshape(-1, target_pc.shape[-1])
    m = q.shape[0]
    n = r.shape[0]
    bm = min(_BM, m)
    grid = m // bm
    rt = r.T

    nums, cnts = pl.pallas_call(
        _knn_body,
        grid=(grid,),
        in_specs=[
            pl.BlockSpec((bm, 3), lambda i: (i, 0)),
            pl.BlockSpec((3, n), lambda i: (0, 0)),
        ],
        out_specs=[
            pl.BlockSpec((1, 1, 1), lambda i: (i, 0, 0)),
            pl.BlockSpec((1, 1, 1), lambda i: (i, 0, 0)),
        ],
        out_shape=[
            jax.ShapeDtypeStruct((grid, 1, 1), jnp.float32),
            jax.ShapeDtypeStruct((grid, 1, 1), jnp.float32),
        ],
        scratch_shapes=[pltpu.VMEM((bm, n), jnp.float32)],
    )(q, rt)

    num = jnp.sum(nums)
    den = jnp.sum(cnts) * float(_K)
    return num / den


# speedup vs baseline: 32.0684x; 2.2787x over previous
import jax
import jax.numpy as jnp
from jax import lax
from jax.experimental import pallas as pl
from jax.experimental.pallas import tpu as pltpu

_K = 16
_BM = 256
_NSL = 16

_NEG_INF = float("-inf")
_POS_INF = float("inf")


def _knn_body(q_ref, rt_ref, num_ref, cnt_ref, d_scr, c_scr, u_scr,
              done_scr):
    n = rt_ref.shape[1]
    bm = q_ref.shape[0]
    ng = n // _NSL

    qx = q_ref[:, 0:1]
    qy = q_ref[:, 1:2]
    qz = q_ref[:, 2:3]
    q2 = qx * qx + qy * qy + qz * qz
    qvalid = (qx != 0.0) | (qy != 0.0) | (qz != 0.0)

    bf = jnp.bfloat16
    f32 = jnp.float32
    qxb = qx.astype(bf).astype(f32)
    qyb = qy.astype(bf).astype(f32)
    qzb = qz.astype(bf).astype(f32)

    cmin = jnp.full((bm, ng), _POS_INF, f32)
    for j in range(_NSL):
        sl = pl.ds(j * ng, ng)
        rx = rt_ref[0:1, sl]
        ry = rt_ref[1:2, sl]
        rz = rt_ref[2:3, sl]
        r2 = rx * rx + ry * ry + rz * rz
        rvalid = (rx != 0.0) | (ry != 0.0) | (rz != 0.0)
        rxb = rx.astype(bf).astype(f32)
        ryb = ry.astype(bf).astype(f32)
        rzb = rz.astype(bf).astype(f32)
        d2 = (q2 + r2) - 2.0 * (qxb * rxb + qyb * ryb + qzb * rzb)
        d2 = jnp.where(rvalid, d2, _POS_INF)
        d_scr[:, sl] = d2
        cmin = jnp.minimum(cmin, d2)
    c_scr[:, :] = cmin

    def c_step(_, t0):
        c = c_scr[:, :]
        v = jnp.min(c, axis=1, keepdims=True)
        c_scr[:, :] = jnp.where(c == v, _POS_INF, c)
        return v

    t0 = lax.fori_loop(0, _K, c_step, jnp.zeros((bm, 1), f32))

    d_all = d_scr[:, :]
    u_scr[:, :] = jnp.max(jnp.where(d_all <= t0, d_all, _NEG_INF), axis=1,
                          keepdims=True)
    done_scr[:, :] = jnp.zeros((bm, 1), f32)

    def w_cond(nd):
        return nd > 0.0

    def w_body(_):
        u = u_scr[:, :]
        done = done_scr[:, :] > 0.0
        d = d_scr[:, :]
        lt = d < u
        cntlt = jnp.sum(lt.astype(f32), axis=1, keepdims=True)
        nxt = jnp.max(jnp.where(lt, d, _NEG_INF), axis=1, keepdims=True)
        done_new = done | (cntlt < float(_K))
        u_scr[:, :] = jnp.where(done_new, u, nxt)
        done_scr[:, :] = done_new.astype(f32)
        return jnp.sum(f32(1.0) - done_new.astype(f32))

    lax.while_loop(w_cond, w_body, jnp.asarray(float(bm), f32))
    t = u_scr[:, :]

    d = d_scr[:, :]
    lt = d < t
    cnt_lt = jnp.sum(lt.astype(f32), axis=1, keepdims=True)
    s_lt = jnp.sum(jnp.where(lt, jnp.sqrt(jnp.maximum(d, 0.0)), 0.0),
                   axis=1, keepdims=True)
    acc = s_lt + (float(_K) - cnt_lt) * jnp.sqrt(jnp.maximum(t, 0.0))

    num_ref[...] = jnp.broadcast_to(jnp.sum(jnp.where(qvalid, acc, 0.0)),
                                    (1, 1, 1))
    cnt_ref[...] = jnp.broadcast_to(jnp.sum(qvalid.astype(f32)), (1, 1, 1))


def kernel(source_pc, target_pc):
    q = source_pc.reshape(-1, source_pc.shape[-1])
    r = target_pc.reshape(-1, target_pc.shape[-1])
    m = q.shape[0]
    n = r.shape[0]
    bm = min(_BM, m)
    grid = m // bm
    rt = r.T

    nums, cnts = pl.pallas_call(
        _knn_body,
        grid=(grid,),
        in_specs=[
            pl.BlockSpec((bm, 3), lambda i: (i, 0)),
            pl.BlockSpec((3, n), lambda i: (0, 0)),
        ],
        out_specs=[
            pl.BlockSpec((1, 1, 1), lambda i: (i, 0, 0)),
            pl.BlockSpec((1, 1, 1), lambda i: (i, 0, 0)),
        ],
        out_shape=[
            jax.ShapeDtypeStruct((grid, 1, 1), jnp.float32),
            jax.ShapeDtypeStruct((grid, 1, 1), jnp.float32),
        ],
        scratch_shapes=[
            pltpu.VMEM((bm, n), jnp.float32),
            pltpu.VMEM((bm, n // _NSL), jnp.float32),
            pltpu.VMEM((bm, 1), jnp.float32),
            pltpu.VMEM((bm, 1), jnp.float32),
        ],
    )(q, rt)

    num = jnp.sum(nums)
    den = jnp.sum(cnts) * float(_K)
    return num / den


# speedup vs baseline: 34.8858x; 1.0879x over previous
import jax
import jax.numpy as jnp
from jax import lax
from jax.experimental import pallas as pl
from jax.experimental.pallas import tpu as pltpu

_K = 16
_BM = 256
_NSL = 16
_NLVL = 4

_NEG_INF = float("-inf")
_POS_INF = float("inf")


def _knn_body(q_ref, rt_ref, num_ref, cnt_ref, b_scr, h_scr, lv_scr, d_scr,
              acc_scr):
    n = rt_ref.shape[1]
    bm = q_ref.shape[0]
    ng = n // _NSL
    f32 = jnp.float32
    bf = jnp.bfloat16

    qx = q_ref[:, 0:1]
    qy = q_ref[:, 1:2]
    qz = q_ref[:, 2:3]
    q2 = qx * qx + qy * qy + qz * qz
    qvalid = (qx != 0.0) | (qy != 0.0) | (qz != 0.0)

    qxb = qx.astype(bf).astype(f32)
    qyb = qy.astype(bf).astype(f32)
    qzb = qz.astype(bf).astype(f32)

    def d2_slice(j):
        sl = pl.ds(j * ng, ng)
        rx = rt_ref[0:1, sl]
        ry = rt_ref[1:2, sl]
        rz = rt_ref[2:3, sl]
        r2 = rx * rx + ry * ry + rz * rz
        rvalid = (rx != 0.0) | (ry != 0.0) | (rz != 0.0)
        rxb = rx.astype(bf).astype(f32)
        ryb = ry.astype(bf).astype(f32)
        rzb = rz.astype(bf).astype(f32)
        d2 = (q2 + r2) - 2.0 * (qxb * rxb + qyb * ryb + qzb * rzb)
        return jnp.where(rvalid, d2, _POS_INF)

    inf_tile = jnp.full((bm, ng), _POS_INF, f32)
    r0, r1, r2_, r3 = inf_tile, inf_tile, inf_tile, inf_tile
    for j in range(_NSL):
        s = d2_slice(j)
        n0 = jnp.minimum(r0, s)
        m0 = jnp.maximum(r0, s)
        n1 = jnp.minimum(r1, m0)
        m1 = jnp.maximum(r1, m0)
        n2 = jnp.minimum(r2_, m1)
        m2 = jnp.maximum(r2_, m1)
        n3 = jnp.minimum(r3, m2)
        r0, r1, r2_, r3 = n0, n1, n2, n3
    b_scr[0] = r0
    b_scr[1] = r1
    b_scr[2] = r2_
    b_scr[3] = r3

    h_scr[:, :] = r0
    lv_scr[:, :] = jnp.zeros((bm, ng), f32)

    def ex_step(_, carry):
        acc, rem = carry
        h = h_scr[:, :]
        v = jnp.min(h, axis=1, keepdims=True)
        eq = h == v
        cnt = jnp.sum(eq.astype(f32), axis=1, keepdims=True)
        take = jnp.minimum(cnt, rem)
        dist = jnp.sqrt(jnp.maximum(v, 0.0))
        acc = acc + jnp.where(take > 0.0, take * dist, 0.0)
        lv = lv_scr[:, :] + eq.astype(f32)
        nxt = jnp.where(lv == 1.0, b_scr[1],
                        jnp.where(lv == 2.0, b_scr[2],
                                  jnp.where(lv == 3.0, b_scr[3], _POS_INF)))
        h_scr[:, :] = jnp.where(eq, nxt, h)
        lv_scr[:, :] = lv
        return acc, rem - take

    acc0 = jnp.zeros((bm, 1), f32)
    rem0 = jnp.full((bm, 1), float(_K), f32)
    acc_fast, _ = lax.fori_loop(0, _K, ex_step, (acc0, rem0))
    acc_scr[:, :] = acc_fast

    flag = jnp.max((lv_scr[:, :] >= float(_NLVL)).astype(f32))

    @pl.when(flag > 0.0)
    def fallback():
        for j in range(_NSL):
            d_scr[:, pl.ds(j * ng, ng)] = d2_slice(j)

        def fb_step(_, carry):
            acc, rem = carry
            d = d_scr[:, :]
            v = jnp.min(d, axis=1, keepdims=True)
            eq = d == v
            c = jnp.sum(eq.astype(f32), axis=1, keepdims=True)
            take = jnp.minimum(c, rem)
            dist = jnp.sqrt(jnp.maximum(v, 0.0))
            acc = acc + jnp.where(take > 0.0, take * dist, 0.0)
            d_scr[:, :] = jnp.where(eq, _POS_INF, d)
            return acc, rem - take

        acc_fb, _ = lax.fori_loop(0, _K, fb_step, (acc0, rem0))
        acc_scr[:, :] = acc_fb

    acc = acc_scr[:, :]

    num_ref[...] = jnp.broadcast_to(jnp.sum(jnp.where(qvalid, acc, 0.0)),
                                    (1, 1, 1))
    cnt_ref[...] = jnp.broadcast_to(jnp.sum(qvalid.astype(f32)), (1, 1, 1))


def kernel(source_pc, target_pc):
    q = source_pc.reshape(-1, source_pc.shape[-1])
    r = target_pc.reshape(-1, target_pc.shape[-1])
    m = q.shape[0]
    n = r.shape[0]
    bm = min(_BM, m)
    grid = m // bm
    rt = r.T

    nums, cnts = pl.pallas_call(
        _knn_body,
        grid=(grid,),
        in_specs=[
            pl.BlockSpec((bm, 3), lambda i: (i, 0)),
            pl.BlockSpec((3, n), lambda i: (0, 0)),
        ],
        out_specs=[
            pl.BlockSpec((1, 1, 1), lambda i: (i, 0, 0)),
            pl.BlockSpec((1, 1, 1), lambda i: (i, 0, 0)),
        ],
        out_shape=[
            jax.ShapeDtypeStruct((grid, 1, 1), jnp.float32),
            jax.ShapeDtypeStruct((grid, 1, 1), jnp.float32),
        ],
        scratch_shapes=[
            pltpu.VMEM((_NLVL, bm, n // _NSL), jnp.float32),
            pltpu.VMEM((bm, n // _NSL), jnp.float32),
            pltpu.VMEM((bm, n // _NSL), jnp.float32),
            pltpu.VMEM((bm, n), jnp.float32),
            pltpu.VMEM((bm, 1), jnp.float32),
        ],
    )(q, rt)

    num = jnp.sum(nums)
    den = jnp.sum(cnts) * float(_K)
    return num / den


# speedup vs baseline: 52.7628x; 1.5124x over previous
import jax
import jax.numpy as jnp
from jax import lax
from jax.experimental import pallas as pl
from jax.experimental.pallas import tpu as pltpu

_K = 16
_BM = 256
_NSL = 16
_NLVL = 4

_NEG_INF = float("-inf")
_POS_INF = float("inf")


def _knn_body(q_ref, rt_ref, qb_ref, rtb_ref, num_ref, cnt_ref, b_scr, h_scr,
              lv_scr, d_scr, acc_scr):
    n = rt_ref.shape[1]
    bm = q_ref.shape[0]
    ng = n // _NSL
    f32 = jnp.float32

    qx = q_ref[:, 0:1]
    qy = q_ref[:, 1:2]
    qz = q_ref[:, 2:3]
    q2 = qx * qx + qy * qy + qz * qz
    qvalid = (qx != 0.0) | (qy != 0.0) | (qz != 0.0)

    qb = qb_ref[:, :]

    def d2_slice(j):
        sl = pl.ds(j * ng, ng)
        rx = rt_ref[0:1, sl]
        ry = rt_ref[1:2, sl]
        rz = rt_ref[2:3, sl]
        r2 = rx * rx + ry * ry + rz * rz
        rvalid = (rx != 0.0) | (ry != 0.0) | (rz != 0.0)
        s = jax.lax.dot_general(qb, rtb_ref[:, sl],
                                (((1,), (0,)), ((), ())),
                                preferred_element_type=f32)
        d2 = (q2 + r2) - 2.0 * s
        return jnp.where(rvalid, d2, _POS_INF)

    inf_tile = jnp.full((bm, ng), _POS_INF, f32)
    r0, r1, r2_, r3 = inf_tile, inf_tile, inf_tile, inf_tile
    for j in range(_NSL):
        s = d2_slice(j)
        n0 = jnp.minimum(r0, s)
        m0 = jnp.maximum(r0, s)
        n1 = jnp.minimum(r1, m0)
        m1 = jnp.maximum(r1, m0)
        n2 = jnp.minimum(r2_, m1)
        m2 = jnp.maximum(r2_, m1)
        n3 = jnp.minimum(r3, m2)
        r0, r1, r2_, r3 = n0, n1, n2, n3
    b_scr[0] = r0
    b_scr[1] = r1
    b_scr[2] = r2_
    b_scr[3] = r3

    h_scr[:, :] = r0
    lv_scr[:, :] = jnp.zeros((bm, ng), f32)

    def ex_step(_, carry):
        acc, rem = carry
        h = h_scr[:, :]
        v = jnp.min(h, axis=1, keepdims=True)
        eq = h == v
        cnt = jnp.sum(eq.astype(f32), axis=1, keepdims=True)
        take = jnp.minimum(cnt, rem)
        dist = jnp.sqrt(jnp.maximum(v, 0.0))
        acc = acc + jnp.where(take > 0.0, take * dist, 0.0)
        lv = lv_scr[:, :] + eq.astype(f32)
        nxt = jnp.where(lv == 1.0, b_scr[1],
                        jnp.where(lv == 2.0, b_scr[2],
                                  jnp.where(lv == 3.0, b_scr[3], _POS_INF)))
        h_scr[:, :] = jnp.where(eq, nxt, h)
        lv_scr[:, :] = lv
        return acc, rem - take

    acc0 = jnp.zeros((bm, 1), f32)
    rem0 = jnp.full((bm, 1), float(_K), f32)
    acc_fast, _ = lax.fori_loop(0, _K, ex_step, (acc0, rem0))
    acc_scr[:, :] = acc_fast

    flag = jnp.max((lv_scr[:, :] >= float(_NLVL)).astype(f32))

    @pl.when(flag > 0.0)
    def fallback():
        for j in range(_NSL):
            d_scr[:, pl.ds(j * ng, ng)] = d2_slice(j)

        def fb_step(_, carry):
            acc, rem = carry
            d = d_scr[:, :]
            v = jnp.min(d, axis=1, keepdims=True)
            eq = d == v
            c = jnp.sum(eq.astype(f32), axis=1, keepdims=True)
            take = jnp.minimum(c, rem)
            dist = jnp.sqrt(jnp.maximum(v, 0.0))
            acc = acc + jnp.where(take > 0.0, take * dist, 0.0)
            d_scr[:, :] = jnp.where(eq, _POS_INF, d)
            return acc, rem - take

        acc_fb, _ = lax.fori_loop(0, _K, fb_step, (acc0, rem0))
        acc_scr[:, :] = acc_fb

    acc = acc_scr[:, :]

    num_ref[...] = jnp.broadcast_to(jnp.sum(jnp.where(qvalid, acc, 0.0)),
                                    (1, 1, 1))
    cnt_ref[...] = jnp.broadcast_to(jnp.sum(qvalid.astype(f32)), (1, 1, 1))


def kernel(source_pc, target_pc):
    q = source_pc.reshape(-1, source_pc.shape[-1])
    r = target_pc.reshape(-1, target_pc.shape[-1])
    m = q.shape[0]
    n = r.shape[0]
    bm = min(_BM, m)
    grid = m // bm
    rt = r.T
    qb = jnp.pad(q, ((0, 0), (0, 5))).astype(jnp.bfloat16)
    rtb = jnp.pad(rt, ((0, 5), (0, 0))).astype(jnp.bfloat16)

    nums, cnts = pl.pallas_call(
        _knn_body,
        grid=(grid,),
        in_specs=[
            pl.BlockSpec((bm, 3), lambda i: (i, 0)),
            pl.BlockSpec((3, n), lambda i: (0, 0)),
            pl.BlockSpec((bm, 8), lambda i: (i, 0)),
            pl.BlockSpec((8, n), lambda i: (0, 0)),
        ],
        out_specs=[
            pl.BlockSpec((1, 1, 1), lambda i: (i, 0, 0)),
            pl.BlockSpec((1, 1, 1), lambda i: (i, 0, 0)),
        ],
        out_shape=[
            jax.ShapeDtypeStruct((grid, 1, 1), jnp.float32),
            jax.ShapeDtypeStruct((grid, 1, 1), jnp.float32),
        ],
        scratch_shapes=[
            pltpu.VMEM((_NLVL, bm, n // _NSL), jnp.float32),
            pltpu.VMEM((bm, n // _NSL), jnp.float32),
            pltpu.VMEM((bm, n // _NSL), jnp.float32),
            pltpu.VMEM((bm, n), jnp.float32),
            pltpu.VMEM((bm, 1), jnp.float32),
        ],
    )(q, rt, qb, rtb)

    num = jnp.sum(nums)
    den = jnp.sum(cnts) * float(_K)
    return num / den


# speedup vs baseline: 54.1682x; 1.0266x over previous
import jax
import jax.numpy as jnp
from jax import lax
from jax.experimental import pallas as pl
from jax.experimental.pallas import tpu as pltpu

_K = 16
_BM = 256
_NSL = 16
_NLVL = 4

_NEG_INF = float("-inf")
_POS_INF = float("inf")


def _knn_body(q_ref, rt_ref, qb_ref, rtb_ref, num_ref, cnt_ref, b_scr, h_scr,
              lv_scr, d_scr, acc_scr, r2p_scr):
    n = rt_ref.shape[1]
    bm = q_ref.shape[0]
    ng = n // _NSL
    f32 = jnp.float32

    qx = q_ref[:, 0:1]
    qy = q_ref[:, 1:2]
    qz = q_ref[:, 2:3]
    q2 = qx * qx + qy * qy + qz * qz
    qvalid = (qx != 0.0) | (qy != 0.0) | (qz != 0.0)

    qb = qb_ref[:, :]

    @pl.when(pl.program_id(0) == 0)
    def _init_r2p():
        rx = rt_ref[0:1, :]
        ry = rt_ref[1:2, :]
        rz = rt_ref[2:3, :]
        r2 = rx * rx + ry * ry + rz * rz
        rvalid = (rx != 0.0) | (ry != 0.0) | (rz != 0.0)
        r2p_scr[:, :] = jnp.where(rvalid, r2, _POS_INF)

    def d2_slice(j):
        sl = pl.ds(j * ng, ng)
        s2 = jax.lax.dot_general(qb, rtb_ref[:, sl],
                                 (((1,), (0,)), ((), ())),
                                 preferred_element_type=f32)
        return (q2 + r2p_scr[0:1, sl]) - s2

    inf_tile = jnp.full((bm, ng), _POS_INF, f32)
    r0, r1, r2_, r3 = inf_tile, inf_tile, inf_tile, inf_tile
    for j in range(_NSL):
        s = d2_slice(j)
        n0 = jnp.minimum(r0, s)
        m0 = jnp.maximum(r0, s)
        n1 = jnp.minimum(r1, m0)
        m1 = jnp.maximum(r1, m0)
        n2 = jnp.minimum(r2_, m1)
        m2 = jnp.maximum(r2_, m1)
        n3 = jnp.minimum(r3, m2)
        r0, r1, r2_, r3 = n0, n1, n2, n3
    b_scr[0] = r0
    b_scr[1] = r1
    b_scr[2] = r2_
    b_scr[3] = r3

    h_scr[:, :] = r0
    lv_scr[:, :] = jnp.zeros((bm, ng), f32)

    def ex_step(_, carry):
        acc, rem = carry
        h = h_scr[:, :]
        v = jnp.min(h, axis=1, keepdims=True)
        eq = h == v
        cnt = jnp.sum(eq.astype(f32), axis=1, keepdims=True)
        take = jnp.minimum(cnt, rem)
        dist = jnp.sqrt(jnp.maximum(v, 0.0))
        acc = acc + jnp.where(take > 0.0, take * dist, 0.0)
        lv = lv_scr[:, :] + eq.astype(f32)
        nxt = jnp.where(lv == 1.0, b_scr[1],
                        jnp.where(lv == 2.0, b_scr[2],
                                  jnp.where(lv == 3.0, b_scr[3], _POS_INF)))
        h_scr[:, :] = jnp.where(eq, nxt, h)
        lv_scr[:, :] = lv
        return acc, rem - take

    acc0 = jnp.zeros((bm, 1), f32)
    rem0 = jnp.full((bm, 1), float(_K), f32)
    acc_fast, _ = lax.fori_loop(0, _K, ex_step, (acc0, rem0))
    acc_scr[:, :] = acc_fast

    flag = jnp.max((lv_scr[:, :] >= float(_NLVL)).astype(f32))

    @pl.when(flag > 0.0)
    def fallback():
        for j in range(_NSL):
            d_scr[:, pl.ds(j * ng, ng)] = d2_slice(j)

        def fb_step(_, carry):
            acc, rem = carry
            d = d_scr[:, :]
            v = jnp.min(d, axis=1, keepdims=True)
            eq = d == v
            c = jnp.sum(eq.astype(f32), axis=1, keepdims=True)
            take = jnp.minimum(c, rem)
            dist = jnp.sqrt(jnp.maximum(v, 0.0))
            acc = acc + jnp.where(take > 0.0, take * dist, 0.0)
            d_scr[:, :] = jnp.where(eq, _POS_INF, d)
            return acc, rem - take

        acc_fb, _ = lax.fori_loop(0, _K, fb_step, (acc0, rem0))
        acc_scr[:, :] = acc_fb

    acc = acc_scr[:, :]

    num_ref[...] = jnp.broadcast_to(jnp.sum(jnp.where(qvalid, acc, 0.0)),
                                    (1, 1, 1))
    cnt_ref[...] = jnp.broadcast_to(jnp.sum(qvalid.astype(f32)), (1, 1, 1))


def kernel(source_pc, target_pc):
    q = source_pc.reshape(-1, source_pc.shape[-1])
    r = target_pc.reshape(-1, target_pc.shape[-1])
    m = q.shape[0]
    n = r.shape[0]
    bm = min(_BM, m)
    grid = m // bm
    rt = r.T
    qb = jnp.pad(2.0 * q, ((0, 0), (0, 5))).astype(jnp.bfloat16)
    rtb = jnp.pad(rt, ((0, 5), (0, 0))).astype(jnp.bfloat16)

    nums, cnts = pl.pallas_call(
        _knn_body,
        grid=(grid,),
        in_specs=[
            pl.BlockSpec((bm, 3), lambda i: (i, 0)),
            pl.BlockSpec((3, n), lambda i: (0, 0)),
            pl.BlockSpec((bm, 8), lambda i: (i, 0)),
            pl.BlockSpec((8, n), lambda i: (0, 0)),
        ],
        out_specs=[
            pl.BlockSpec((1, 1, 1), lambda i: (i, 0, 0)),
            pl.BlockSpec((1, 1, 1), lambda i: (i, 0, 0)),
        ],
        out_shape=[
            jax.ShapeDtypeStruct((grid, 1, 1), jnp.float32),
            jax.ShapeDtypeStruct((grid, 1, 1), jnp.float32),
        ],
        scratch_shapes=[
            pltpu.VMEM((_NLVL, bm, n // _NSL), jnp.float32),
            pltpu.VMEM((bm, n // _NSL), jnp.float32),
            pltpu.VMEM((bm, n // _NSL), jnp.float32),
            pltpu.VMEM((bm, n), jnp.float32),
            pltpu.VMEM((bm, 1), jnp.float32),
            pltpu.VMEM((1, n), jnp.float32),
        ],
    )(q, rt, qb, rtb)

    num = jnp.sum(nums)
    den = jnp.sum(cnts) * float(_K)
    return num / den


# speedup vs baseline: 60.4737x; 1.1164x over previous
import jax
import jax.numpy as jnp
from jax import lax
from jax.experimental import pallas as pl
from jax.experimental.pallas import tpu as pltpu

_K = 16
_BM = 512
_NSL = 16
_NLVL = 4

_NEG_INF = float("-inf")
_POS_INF = float("inf")


def _knn_body(q_ref, rt_ref, qb_ref, rtb_ref, num_ref, cnt_ref, b_scr, h_scr,
              lv_scr, d_scr, acc_scr, r2p_scr):
    n = rt_ref.shape[1]
    bm = q_ref.shape[0]
    ng = n // _NSL
    f32 = jnp.float32

    qx = q_ref[:, 0:1]
    qy = q_ref[:, 1:2]
    qz = q_ref[:, 2:3]
    q2 = qx * qx + qy * qy + qz * qz
    qvalid = (qx != 0.0) | (qy != 0.0) | (qz != 0.0)

    qb = qb_ref[:, :]

    @pl.when(pl.program_id(0) == 0)
    def _init_r2p():
        rx = rt_ref[0:1, :]
        ry = rt_ref[1:2, :]
        rz = rt_ref[2:3, :]
        r2 = rx * rx + ry * ry + rz * rz
        rvalid = (rx != 0.0) | (ry != 0.0) | (rz != 0.0)
        r2p_scr[:, :] = jnp.where(rvalid, r2, _POS_INF)

    def d2_slice(j):
        sl = pl.ds(j * ng, ng)
        s2 = jax.lax.dot_general(qb, rtb_ref[:, sl],
                                 (((1,), (0,)), ((), ())),
                                 preferred_element_type=f32)
        return (q2 + r2p_scr[0:1, sl]) - s2

    inf_tile = jnp.full((bm, ng), _POS_INF, f32)
    r0, r1, r2_, r3 = inf_tile, inf_tile, inf_tile, inf_tile
    for j in range(_NSL):
        s = d2_slice(j)
        n0 = jnp.minimum(r0, s)
        m0 = jnp.maximum(r0, s)
        n1 = jnp.minimum(r1, m0)
        m1 = jnp.maximum(r1, m0)
        n2 = jnp.minimum(r2_, m1)
        m2 = jnp.maximum(r2_, m1)
        n3 = jnp.minimum(r3, m2)
        r0, r1, r2_, r3 = n0, n1, n2, n3
    b_scr[0] = r0
    b_scr[1] = r1
    b_scr[2] = r2_
    b_scr[3] = r3

    h_scr[:, :] = r0
    lv_scr[:, :] = jnp.zeros((bm, ng), f32)

    def ex_step(_, carry):
        acc, rem = carry
        h = h_scr[:, :]
        v = jnp.min(h, axis=1, keepdims=True)
        eq = h == v
        cnt = jnp.sum(eq.astype(f32), axis=1, keepdims=True)
        take = jnp.minimum(cnt, rem)
        dist = jnp.sqrt(jnp.maximum(v, 0.0))
        acc = acc + jnp.where(take > 0.0, take * dist, 0.0)
        lv = lv_scr[:, :] + eq.astype(f32)
        nxt = jnp.where(lv == 1.0, b_scr[1],
                        jnp.where(lv == 2.0, b_scr[2],
                                  jnp.where(lv == 3.0, b_scr[3], _POS_INF)))
        h_scr[:, :] = jnp.where(eq, nxt, h)
        lv_scr[:, :] = lv
        return acc, rem - take

    acc0 = jnp.zeros((bm, 1), f32)
    rem0 = jnp.full((bm, 1), float(_K), f32)
    acc_fast, _ = lax.fori_loop(0, _K, ex_step, (acc0, rem0))
    acc_scr[:, :] = acc_fast

    flag = jnp.max((lv_scr[:, :] >= float(_NLVL)).astype(f32))

    @pl.when(flag > 0.0)
    def fallback():
        for j in range(_NSL):
            d_scr[:, pl.ds(j * ng, ng)] = d2_slice(j)

        def fb_step(_, carry):
            acc, rem = carry
            d = d_scr[:, :]
            v = jnp.min(d, axis=1, keepdims=True)
            eq = d == v
            c = jnp.sum(eq.astype(f32), axis=1, keepdims=True)
            take = jnp.minimum(c, rem)
            dist = jnp.sqrt(jnp.maximum(v, 0.0))
            acc = acc + jnp.where(take > 0.0, take * dist, 0.0)
            d_scr[:, :] = jnp.where(eq, _POS_INF, d)
            return acc, rem - take

        acc_fb, _ = lax.fori_loop(0, _K, fb_step, (acc0, rem0))
        acc_scr[:, :] = acc_fb

    acc = acc_scr[:, :]

    num_ref[...] = jnp.broadcast_to(jnp.sum(jnp.where(qvalid, acc, 0.0)),
                                    (1, 1, 1))
    cnt_ref[...] = jnp.broadcast_to(jnp.sum(qvalid.astype(f32)), (1, 1, 1))


def kernel(source_pc, target_pc):
    q = source_pc.reshape(-1, source_pc.shape[-1])
    r = target_pc.reshape(-1, target_pc.shape[-1])
    m = q.shape[0]
    n = r.shape[0]
    bm = min(_BM, m)
    grid = m // bm
    rt = r.T
    qb = jnp.pad(2.0 * q, ((0, 0), (0, 5))).astype(jnp.bfloat16)
    rtb = jnp.pad(rt, ((0, 5), (0, 0))).astype(jnp.bfloat16)

    nums, cnts = pl.pallas_call(
        _knn_body,
        grid=(grid,),
        in_specs=[
            pl.BlockSpec((bm, 3), lambda i: (i, 0)),
            pl.BlockSpec((3, n), lambda i: (0, 0)),
            pl.BlockSpec((bm, 8), lambda i: (i, 0)),
            pl.BlockSpec((8, n), lambda i: (0, 0)),
        ],
        out_specs=[
            pl.BlockSpec((1, 1, 1), lambda i: (i, 0, 0)),
            pl.BlockSpec((1, 1, 1), lambda i: (i, 0, 0)),
        ],
        out_shape=[
            jax.ShapeDtypeStruct((grid, 1, 1), jnp.float32),
            jax.ShapeDtypeStruct((grid, 1, 1), jnp.float32),
        ],
        scratch_shapes=[
            pltpu.VMEM((_NLVL, bm, n // _NSL), jnp.float32),
            pltpu.VMEM((bm, n // _NSL), jnp.float32),
            pltpu.VMEM((bm, n // _NSL), jnp.float32),
            pltpu.VMEM((bm, n), jnp.float32),
            pltpu.VMEM((bm, 1), jnp.float32),
            pltpu.VMEM((1, n), jnp.float32),
        ],
    )(q, rt, qb, rtb)

    num = jnp.sum(nums)
    den = jnp.sum(cnts) * float(_K)
    return num / den


# speedup vs baseline: 65.9595x; 1.0907x over previous
import jax
import jax.numpy as jnp
from jax import lax
from jax.experimental import pallas as pl
from jax.experimental.pallas import tpu as pltpu

_K = 16
_BM = 512
_NSL = 16
_NLVL = 5

_NEG_INF = float("-inf")
_POS_INF = float("inf")


def _knn_body(q_ref, rt_ref, qb_ref, rtb_ref, num_ref, cnt_ref, b_scr, h_scr,
              lv_scr, d_scr, acc_scr, r2p_scr):
    n = rt_ref.shape[1]
    bm = q_ref.shape[0]
    ng = n // _NSL
    f32 = jnp.float32

    qx = q_ref[:, 0:1]
    qy = q_ref[:, 1:2]
    qz = q_ref[:, 2:3]
    q2 = qx * qx + qy * qy + qz * qz
    qvalid = (qx != 0.0) | (qy != 0.0) | (qz != 0.0)

    qb = qb_ref[:, :]

    @pl.when(pl.program_id(0) == 0)
    def _init_r2p():
        rx = rt_ref[0:1, :]
        ry = rt_ref[1:2, :]
        rz = rt_ref[2:3, :]
        r2 = rx * rx + ry * ry + rz * rz
        rvalid = (rx != 0.0) | (ry != 0.0) | (rz != 0.0)
        r2p_scr[:, :] = jnp.where(rvalid, r2, _POS_INF)

    def d2_slice(j):
        sl = pl.ds(j * ng, ng)
        s2 = jax.lax.dot_general(qb, rtb_ref[:, sl],
                                 (((1,), (0,)), ((), ())),
                                 preferred_element_type=f32)
        return (q2 + r2p_scr[0:1, sl]) - s2

    inf_tile = jnp.full((bm, ng), _POS_INF, f32)
    r0, r1, r2_, r3, r4 = (inf_tile,) * 5
    for j in range(_NSL):
        s = d2_slice(j)
        n0 = jnp.minimum(r0, s)
        m0 = jnp.maximum(r0, s)
        n1 = jnp.minimum(r1, m0)
        m1 = jnp.maximum(r1, m0)
        n2 = jnp.minimum(r2_, m1)
        m2 = jnp.maximum(r2_, m1)
        n3 = jnp.minimum(r3, m2)
        m3 = jnp.maximum(r3, m2)
        n4 = jnp.minimum(r4, m3)
        r0, r1, r2_, r3, r4 = n0, n1, n2, n3, n4
    nh = ng // 2
    a = [r0[:, :nh], r1[:, :nh], r2_[:, :nh], r3[:, :nh], r4[:, :nh]]
    b = [r0[:, nh:], r1[:, nh:], r2_[:, nh:], r3[:, nh:], r4[:, nh:]]
    c = [jnp.minimum(a[i], b[4 - i]) for i in range(5)]
    for i, j in ((0, 3), (1, 4), (0, 2), (1, 3), (0, 1), (2, 4), (1, 2),
                 (3, 4), (2, 3)):
        lo = jnp.minimum(c[i], c[j])
        hi = jnp.maximum(c[i], c[j])
        c[i], c[j] = lo, hi
    for i in range(5):
        b_scr[i] = c[i]

    h_scr[:, :] = c[0]
    lv_scr[:, :] = jnp.zeros((bm, nh), f32)

    def ex_step(_, carry):
        acc, rem = carry
        h = h_scr[:, :]
        v = jnp.min(h, axis=1, keepdims=True)
        eq = h == v
        cnt = jnp.sum(eq.astype(f32), axis=1, keepdims=True)
        take = jnp.minimum(cnt, rem)
        dist = jnp.sqrt(jnp.maximum(v, 0.0))
        acc = acc + jnp.where(take > 0.0, take * dist, 0.0)
        lv = lv_scr[:, :] + eq.astype(f32)
        nxt = jnp.where(lv == 1.0, b_scr[1],
                        jnp.where(lv == 2.0, b_scr[2],
                                  jnp.where(lv == 3.0, b_scr[3],
                                            jnp.where(lv == 4.0, b_scr[4],
                                                      _POS_INF))))
        h_scr[:, :] = jnp.where(eq, nxt, h)
        lv_scr[:, :] = lv
        return acc, rem - take

    acc0 = jnp.zeros((bm, 1), f32)
    rem0 = jnp.full((bm, 1), float(_K), f32)
    acc_fast, _ = lax.fori_loop(0, _K, ex_step, (acc0, rem0))
    acc_scr[:, :] = acc_fast

    flag = jnp.max((lv_scr[:, :] >= float(_NLVL)).astype(f32))

    @pl.when(flag > 0.0)
    def fallback():
        for j in range(_NSL):
            d_scr[:, pl.ds(j * ng, ng)] = d2_slice(j)

        def fb_step(_, carry):
            acc, rem = carry
            d = d_scr[:, :]
            v = jnp.min(d, axis=1, keepdims=True)
            eq = d == v
            c = jnp.sum(eq.astype(f32), axis=1, keepdims=True)
            take = jnp.minimum(c, rem)
            dist = jnp.sqrt(jnp.maximum(v, 0.0))
            acc = acc + jnp.where(take > 0.0, take * dist, 0.0)
            d_scr[:, :] = jnp.where(eq, _POS_INF, d)
            return acc, rem - take

        acc_fb, _ = lax.fori_loop(0, _K, fb_step, (acc0, rem0))
        acc_scr[:, :] = acc_fb

    acc = acc_scr[:, :]

    num_ref[...] = jnp.broadcast_to(jnp.sum(jnp.where(qvalid, acc, 0.0)),
                                    (1, 1, 1))
    cnt_ref[...] = jnp.broadcast_to(jnp.sum(qvalid.astype(f32)), (1, 1, 1))


def kernel(source_pc, target_pc):
    q = source_pc.reshape(-1, source_pc.shape[-1])
    r = target_pc.reshape(-1, target_pc.shape[-1])
    m = q.shape[0]
    n = r.shape[0]
    bm = min(_BM, m)
    grid = m // bm
    rt = r.T
    qb = jnp.pad(2.0 * q, ((0, 0), (0, 5))).astype(jnp.bfloat16)
    rtb = jnp.pad(rt, ((0, 5), (0, 0))).astype(jnp.bfloat16)

    nums, cnts = pl.pallas_call(
        _knn_body,
        grid=(grid,),
        in_specs=[
            pl.BlockSpec((bm, 3), lambda i: (i, 0)),
            pl.BlockSpec((3, n), lambda i: (0, 0)),
            pl.BlockSpec((bm, 8), lambda i: (i, 0)),
            pl.BlockSpec((8, n), lambda i: (0, 0)),
        ],
        out_specs=[
            pl.BlockSpec((1, 1, 1), lambda i: (i, 0, 0)),
            pl.BlockSpec((1, 1, 1), lambda i: (i, 0, 0)),
        ],
        out_shape=[
            jax.ShapeDtypeStruct((grid, 1, 1), jnp.float32),
            jax.ShapeDtypeStruct((grid, 1, 1), jnp.float32),
        ],
        scratch_shapes=[
            pltpu.VMEM((_NLVL, bm, n // _NSL // 2), jnp.float32),
            pltpu.VMEM((bm, n // _NSL // 2), jnp.float32),
            pltpu.VMEM((bm, n // _NSL // 2), jnp.float32),
            pltpu.VMEM((bm, n), jnp.float32),
            pltpu.VMEM((bm, 1), jnp.float32),
            pltpu.VMEM((1, n), jnp.float32),
        ],
    )(q, rt, qb, rtb)

    num = jnp.sum(nums)
    den = jnp.sum(cnts) * float(_K)
    return num / den
